# baseline (speedup 1.0000x reference)
import functools

import jax
import jax.numpy as jnp
from jax import lax
from jax.experimental import pallas as pl
from jax.experimental.pallas import tpu as pltpu
from jax.experimental.pallas import tpu_sc as plsc
_L = 512
_MIN_DIST = 4
_NS = 16
_NROWS = _L // _NS
_NCH = _L // 16


def _mask_kernel(con_ref, seq_ref, out_ref):
    f32 = jnp.float32
    c = con_ref[...]
    c = (c + c.T) * 0.5

    ri = lax.broadcasted_iota(jnp.int32, (_L, _L), 0)
    ci = lax.broadcasted_iota(jnp.int32, (_L, _L), 1)
    band = jnp.abs(ri - ci) >= _MIN_DIST

    s = seq_ref[...]
    best = s[0:1, :]
    prime = jnp.full((1, _L), 2.0, f32)
    for k, p in ((1, 3.0), (2, 5.0), (3, 7.0)):
        sk = s[k : k + 1, :]
        upd = sk > best
        best = jnp.where(upd, sk, best)
        prime = jnp.where(upd, p, prime)
    pcols = jnp.broadcast_to(prime, (_L, _L))
    prows = pcols.T
    prod = pcols * prows
    pmask = (prod == 14.0) | (prod == 15.0) | (prod == 35.0)

    conm = jnp.where(band & pmask, c, 0.0)
    out_ref[...] = jnp.where(conm > 0.0, conm, 0.0)


def _greedy_sc_body(a_hbm, out_hbm, av, outv, alive, rmaxv, argv, strm, starg,
                    partner, rmax_sh, arg_sh):
    cid = lax.axis_index("c")
    sid = lax.axis_index("s")
    base = sid * _NROWS
    lanes = lax.broadcasted_iota(jnp.int32, (16,), 0)
    zeros16 = jnp.zeros((16,), jnp.float32)
    zeros16i = jnp.zeros((16,), jnp.int32)

    pltpu.sync_copy(a_hbm.at[pl.ds(base * _L, _NROWS * _L)], av)

    def _init_alive(k, x):
        alive[pl.ds(k * 16, 16)] = jnp.ones((16,), jnp.float32)
        return x
    lax.fori_loop(0, _NCH, _init_alive, 0)

    def _init_out(k, x):
        outv[pl.ds(k * 16, 16)] = zeros16
        return x
    lax.fori_loop(0, _NROWS * _NCH, _init_out, 0)

    neg1 = jnp.full((16,), -1, jnp.int32)
    for g in range(_NROWS // 16):
        partner[pl.ds(g * 16, 16)] = neg1

    def _round(gmax_prev):
        for g in range(_NROWS // 16):
            def _row(rl, carry):
                rvec, cvec = carry
                rowoff = (g * 16 + rl) * _L

                def _chunk(k, mm):
                    mv, mc = mm
                    v = av[pl.ds(rowoff + k * 16, 16)] * alive[pl.ds(k * 16, 16)]
                    cols = k * 16 + lanes
                    upd = v > mv
                    mv = jnp.where(upd, v, mv)
                    mc = jnp.where(upd, cols, mc)
                    return mv, mc

                mv, mc = lax.fori_loop(0, _NCH, _chunk, (zeros16, zeros16i))
                for sh in (8, 4, 2, 1):
                    idx = jnp.bitwise_xor(lanes, sh)
                    pv = jnp.take_along_axis(mv, idx, axis=0, mode="promise_in_bounds")
                    pc = jnp.take_along_axis(mc, idx, axis=0, mode="promise_in_bounds")
                    take = (pv > mv) | ((pv == mv) & (pc < mc))
                    mv = jnp.where(take, pv, mv)
                    mc = jnp.where(take, pc, mc)
                rvec = jnp.where(lanes == rl, mv, rvec)
                cvec = jnp.where(lanes == rl, mc, cvec)
                return rvec, cvec

            rvec, cvec = lax.fori_loop(0, 16, _row, (zeros16, zeros16i))
            rvec = rvec * alive[pl.ds(base + g * 16, 16)]
            strm[...] = rvec
            starg[...] = cvec
            pltpu.sync_copy(strm, rmax_sh.at[pl.ds(base + g * 16, 16)])
            pltpu.sync_copy(starg, arg_sh.at[pl.ds(base + g * 16, 16)])

        plsc.subcore_barrier()
        pltpu.sync_copy(rmax_sh, rmaxv)
        pltpu.sync_copy(arg_sh, argv)
        plsc.subcore_barrier()

        def _sweep(k, gm):
            a = argv[pl.ds(k * 16, 16)]
            rv = rmaxv[pl.ds(k * 16, 16)]
            pa = plsc.load_gather(argv, [a])
            ids = k * 16 + lanes
            mnew = (rv > 0.0) & (pa == ids)
            alive[pl.ds(k * 16, 16)] = jnp.where(
                mnew, 0.0, alive[pl.ds(k * 16, 16)]
            )
            return jnp.maximum(gm, rv)

        gmv = lax.fori_loop(0, _NCH, _sweep, zeros16)
        cont = jnp.any(gmv > 0.0)

        for g in range(_NROWS // 16):
            k = (base // 16) + g
            a = argv[pl.ds(k * 16, 16)]
            rv = rmaxv[pl.ds(k * 16, 16)]
            pa = plsc.load_gather(argv, [a])
            ids = k * 16 + lanes
            mnew = (rv > 0.0) & (pa == ids)
            pc = partner[pl.ds(g * 16, 16)]
            partner[pl.ds(g * 16, 16)] = jnp.where(mnew, a, pc)
        return cont

    def _round_step(r, cont_prev):
        del r
        return lax.cond(cont_prev, _round, lambda c: c, cont_prev)

    lax.fori_loop(0, _L // 2, _round_step, jnp.bool_(True))

    for g in range(_NROWS // 16):
        pc = partner[pl.ds(g * 16, 16)]
        ok = pc >= 0
        pcc = jnp.where(ok, pc, 0)
        flat = (g * 16 + lanes) * _L + pcc
        vals = plsc.load_gather(av, [flat], mask=ok)
        plsc.store_scatter(outv, [flat], vals, mask=ok)

    @pl.when(cid == 0)
    def _():
        pltpu.sync_copy(outv, out_hbm.at[pl.ds(base * _L, _NROWS * _L)])


@functools.cache
def _build_greedy_sc():
    sc_mesh = plsc.VectorSubcoreMesh(
        core_axis_name="c", subcore_axis_name="s", num_cores=2, num_subcores=_NS
    )
    return pl.kernel(
        _greedy_sc_body,
        out_type=jax.ShapeDtypeStruct((_L * _L,), jnp.float32),
        mesh=sc_mesh,
        compiler_params=pltpu.CompilerParams(needs_layout_passes=False),
        scratch_types=[
            pltpu.VMEM((_NROWS * _L,), jnp.float32),
            pltpu.VMEM((_NROWS * _L,), jnp.float32),
            pltpu.VMEM((_L,), jnp.float32),
            pltpu.VMEM((_L,), jnp.float32),
            pltpu.VMEM((_L,), jnp.int32),
            pltpu.VMEM((16,), jnp.float32),
            pltpu.VMEM((16,), jnp.int32),
            pltpu.VMEM((_NROWS,), jnp.int32),
            pltpu.VMEM_SHARED((_L,), jnp.float32),
            pltpu.VMEM_SHARED((_L,), jnp.int32),
        ],
    )


def kernel(con, feat):
    con2 = con.reshape(_L, _L)
    seq = feat[0, :, :, 0]
    a0 = pl.pallas_call(
        _mask_kernel,
        out_shape=jax.ShapeDtypeStruct((_L, _L), jnp.float32),
    )(con2, seq)
    out = _build_greedy_sc()(a0.reshape(_L * _L))
    return out.reshape(con.shape)


# speedup vs baseline: 1.3728x; 1.3728x over previous
import functools

import jax
import jax.numpy as jnp
from jax import lax
from jax.experimental import pallas as pl
from jax.experimental.pallas import tpu as pltpu
from jax.experimental.pallas import tpu_sc as plsc
_L = 512
_MIN_DIST = 4
_NS = 16
_NROWS = _L // _NS
_NCH = _L // 16


def _mask_kernel(con_ref, seq_ref, out_ref):
    f32 = jnp.float32
    c = con_ref[...]
    c = (c + c.T) * 0.5

    ri = lax.broadcasted_iota(jnp.int32, (_L, _L), 0)
    ci = lax.broadcasted_iota(jnp.int32, (_L, _L), 1)
    band = jnp.abs(ri - ci) >= _MIN_DIST

    s = seq_ref[...]
    best = s[0:1, :]
    prime = jnp.full((1, _L), 2.0, f32)
    for k, p in ((1, 3.0), (2, 5.0), (3, 7.0)):
        sk = s[k : k + 1, :]
        upd = sk > best
        best = jnp.where(upd, sk, best)
        prime = jnp.where(upd, p, prime)
    pcols = jnp.broadcast_to(prime, (_L, _L))
    prows = pcols.T
    prod = pcols * prows
    pmask = (prod == 14.0) | (prod == 15.0) | (prod == 35.0)

    conm = jnp.where(band & pmask, c, 0.0)
    out_ref[...] = jnp.where(conm > 0.0, conm, 0.0)


def _greedy_sc_body(a_hbm, out_hbm, av, outv, alive, rmaxv, argv, strm, starg,
                    partner, rmax_sh, arg_sh):
    cid = lax.axis_index("c")
    sid = lax.axis_index("s")
    base = sid * _NROWS
    lanes = lax.broadcasted_iota(jnp.int32, (16,), 0)
    zeros16 = jnp.zeros((16,), jnp.float32)
    zeros16i = jnp.zeros((16,), jnp.int32)

    pltpu.sync_copy(a_hbm.at[pl.ds(base * _L, _NROWS * _L)], av)

    def _init_alive(k, x):
        alive[pl.ds(k * 16, 16)] = jnp.ones((16,), jnp.float32)
        return x
    lax.fori_loop(0, _NCH, _init_alive, 0, unroll=8)

    def _init_out(k, x):
        outv[pl.ds(k * 16, 16)] = zeros16
        return x
    lax.fori_loop(0, _NROWS * _NCH, _init_out, 0, unroll=8)

    neg1 = jnp.full((16,), -1, jnp.int32)
    for g in range(_NROWS // 16):
        partner[pl.ds(g * 16, 16)] = neg1

    def _round(gmax_prev):
        for g in range(_NROWS // 16):
            aliveg = alive[pl.ds(base + g * 16, 16)]

            def _row(rl, carry, aliveg=aliveg, g=g):
                live_r = jnp.any((lanes == rl) & (aliveg > 0.0))

                def _do_row(cr):
                    rvec, cvec = cr
                    rowoff = (g * 16 + rl) * _L

                    def _chunk(k, mm):
                        mv, mc = mm
                        v = av[pl.ds(rowoff + k * 16, 16)] * alive[pl.ds(k * 16, 16)]
                        cols = k * 16 + lanes
                        upd = v > mv
                        mv = jnp.where(upd, v, mv)
                        mc = jnp.where(upd, cols, mc)
                        return mv, mc

                    mv, mc = lax.fori_loop(
                        0, _NCH, _chunk, (zeros16, zeros16i), unroll=4
                    )
                    for sh in (8, 4, 2, 1):
                        idx = jnp.bitwise_xor(lanes, sh)
                        pv = jnp.take_along_axis(mv, idx, axis=0, mode="promise_in_bounds")
                        pc = jnp.take_along_axis(mc, idx, axis=0, mode="promise_in_bounds")
                        take = (pv > mv) | ((pv == mv) & (pc < mc))
                        mv = jnp.where(take, pv, mv)
                        mc = jnp.where(take, pc, mc)
                    rvec = jnp.where(lanes == rl, mv, rvec)
                    cvec = jnp.where(lanes == rl, mc, cvec)
                    return rvec, cvec

                return lax.cond(live_r, _do_row, lambda cr: cr, carry)

            def _do_group(_, g=g):
                return lax.fori_loop(0, 16, _row, (zeros16, zeros16i))

            grp_live = jnp.any(aliveg > 0.0)
            rvec, cvec = lax.cond(
                grp_live, _do_group, lambda _: (zeros16, zeros16i), 0
            )
            strm[pl.ds(g * 16, 16)] = rvec
            starg[pl.ds(g * 16, 16)] = cvec

        pltpu.sync_copy(strm, rmax_sh.at[pl.ds(base, _NROWS)])
        pltpu.sync_copy(starg, arg_sh.at[pl.ds(base, _NROWS)])

        plsc.subcore_barrier()
        pltpu.sync_copy(rmax_sh, rmaxv)
        pltpu.sync_copy(arg_sh, argv)
        plsc.subcore_barrier()

        def _sweep(k, gm):
            rv = rmaxv[pl.ds(k * 16, 16)]

            def _do(gmx):
                a = argv[pl.ds(k * 16, 16)]
                pa = plsc.load_gather(argv, [a])
                ids = k * 16 + lanes
                mnew = (rv > 0.0) & (pa == ids)
                alive[pl.ds(k * 16, 16)] = jnp.where(
                    mnew, 0.0, alive[pl.ds(k * 16, 16)]
                )
                return jnp.maximum(gmx, rv)

            return lax.cond(jnp.any(rv > 0.0), _do, lambda x: x, gm)

        gmv = lax.fori_loop(0, _NCH, _sweep, zeros16)
        cont = jnp.any(gmv > 0.0)

        for g in range(_NROWS // 16):
            k = (base // 16) + g
            rv = rmaxv[pl.ds(k * 16, 16)]

            def _do_rec(x, k=k, g=g, rv=rv):
                a = argv[pl.ds(k * 16, 16)]
                pa = plsc.load_gather(argv, [a])
                ids = k * 16 + lanes
                mnew = (rv > 0.0) & (pa == ids)
                pc = partner[pl.ds(g * 16, 16)]
                partner[pl.ds(g * 16, 16)] = jnp.where(mnew, a, pc)
                return x

            lax.cond(jnp.any(rv > 0.0), _do_rec, lambda x: x, 0)
        return cont

    def _round_step(r, cont_prev):
        del r
        return lax.cond(cont_prev, _round, lambda c: c, cont_prev)

    lax.fori_loop(0, _L // 2, _round_step, jnp.bool_(True))

    for g in range(_NROWS // 16):
        pc = partner[pl.ds(g * 16, 16)]
        ok = pc >= 0
        pcc = jnp.where(ok, pc, 0)
        flat = (g * 16 + lanes) * _L + pcc
        vals = plsc.load_gather(av, [flat], mask=ok)
        plsc.store_scatter(outv, [flat], vals, mask=ok)

    @pl.when(cid == 0)
    def _():
        pltpu.sync_copy(outv, out_hbm.at[pl.ds(base * _L, _NROWS * _L)])


@functools.cache
def _build_greedy_sc():
    sc_mesh = plsc.VectorSubcoreMesh(
        core_axis_name="c", subcore_axis_name="s", num_cores=2, num_subcores=_NS
    )
    return pl.kernel(
        _greedy_sc_body,
        out_type=jax.ShapeDtypeStruct((_L * _L,), jnp.float32),
        mesh=sc_mesh,
        compiler_params=pltpu.CompilerParams(needs_layout_passes=False),
        scratch_types=[
            pltpu.VMEM((_NROWS * _L,), jnp.float32),
            pltpu.VMEM((_NROWS * _L,), jnp.float32),
            pltpu.VMEM((_L,), jnp.float32),
            pltpu.VMEM((_L,), jnp.float32),
            pltpu.VMEM((_L,), jnp.int32),
            pltpu.VMEM((_NROWS,), jnp.float32),
            pltpu.VMEM((_NROWS,), jnp.int32),
            pltpu.VMEM((_NROWS,), jnp.int32),
            pltpu.VMEM_SHARED((_L,), jnp.float32),
            pltpu.VMEM_SHARED((_L,), jnp.int32),
        ],
    )


def kernel(con, feat):
    con2 = con.reshape(_L, _L)
    seq = feat[0, :, :, 0]
    a0 = pl.pallas_call(
        _mask_kernel,
        out_shape=jax.ShapeDtypeStruct((_L, _L), jnp.float32),
    )(con2, seq)
    out = _build_greedy_sc()(a0.reshape(_L * _L))
    return out.reshape(con.shape)
